# baseline (speedup 1.0000x reference)
import functools

import jax
import jax.numpy as jnp
from jax import lax
from jax.experimental import pallas as pl
from jax.experimental.pallas import tpu as pltpu
from jax.experimental.pallas import tpu_sc as plsc

_TOP_K = 8
_E = 64
_LANES = 16


def _matmul_body(x_ref, w_ref, lg_ref):
    lg_ref[...] = jnp.dot(x_ref[...], w_ref[...],
                          preferred_element_type=jnp.float32)


@functools.partial(jax.jit, static_argnames=("block_t",))
def _logits_tc(x, W_gate, block_t=1024):
    T, D = x.shape
    E = W_gate.shape[1]
    nb = T // block_t
    return pl.pallas_call(
        _matmul_body,
        grid=(nb,),
        in_specs=[
            pl.BlockSpec((block_t, D), lambda i: (i, 0)),
            pl.BlockSpec((D, E), lambda i: (0, 0)),
        ],
        out_specs=pl.BlockSpec((block_t, E), lambda i: (i, 0)),
        out_shape=jax.ShapeDtypeStruct((T, E), jnp.float32),
        compiler_params=pltpu.CompilerParams(
            dimension_semantics=("arbitrary",),
        ),
    )(x, W_gate)


def _merge_desc(ka, va, kb, vb):
    rkb = lax.rev(kb, (0,))
    rvb = lax.rev(vb, (0,))
    p = ka >= rkb
    hk = jnp.where(p, ka, rkb)
    hv = jnp.where(p, va, rvb)
    return plsc.sort_key_val(hk, hv, descending=True)


def _make_sc_topk(T, rows_per_chunk):
    n_workers = 32
    rows_per_worker = T // n_workers
    n_chunks = rows_per_worker // rows_per_chunk
    groups_per_chunk = rows_per_chunk // _LANES
    mesh = plsc.VectorSubcoreMesh(core_axis_name="c", subcore_axis_name="s")

    @functools.partial(
        pl.kernel,
        mesh=mesh,
        out_type=[
            jax.ShapeDtypeStruct((T * _E,), jnp.float32),
            jax.ShapeDtypeStruct((T * _TOP_K,), jnp.int32),
        ],
        scratch_types=[
            pltpu.VMEM((rows_per_chunk * _E,), jnp.float32),
            pltpu.VMEM((rows_per_chunk * _E,), jnp.float32),
            pltpu.VMEM((rows_per_chunk * _TOP_K,), jnp.int32),
        ],
        compiler_params=pltpu.CompilerParams(needs_layout_passes=False),
    )
    def sc_topk(lg_hbm, dense_hbm, idx_hbm, lg_v, dense_v, idx_v):
        wid = lax.axis_index("s") * 2 + lax.axis_index("c")
        lane = lax.broadcasted_iota(jnp.int32, (_LANES,), 0)
        low8 = lane < _TOP_K
        zeros16 = jnp.zeros((_LANES,), jnp.float32)

        def do_chunk(ci, carry):
            base = wid * rows_per_worker + ci * rows_per_chunk
            pltpu.sync_copy(lg_hbm.at[pl.ds(base * _E, rows_per_chunk * _E)],
                            lg_v)

            def zero_row(j, c):
                dense_v[pl.ds(j * _LANES, _LANES)] = zeros16
                return c

            lax.fori_loop(0, rows_per_chunk * _E // _LANES, zero_row, 0)

            def do_row(r, c):
                off = r * _E
                ks = []
                vs = []
                for cc in range(4):
                    k = lg_v[pl.ds(off + cc * _LANES, _LANES)]
                    v = lane + cc * _LANES
                    ks_c, vs_c = plsc.sort_key_val(k, v, descending=True)
                    ks.append(ks_c)
                    vs.append(vs_c)
                k01, v01 = _merge_desc(ks[0], vs[0], ks[1], vs[1])
                k23, v23 = _merge_desc(ks[2], vs[2], ks[3], vs[3])
                kf, vf = _merge_desc(k01, v01, k23, v23)
                t0 = lax.reduce_max(kf, (0,))
                e = jnp.where(low8, jnp.exp(kf - t0), 0.0)
                s = lax.reduce_sum(e, (0,))
                gates = e / s
                plsc.store_scatter(dense_v, [off + vf], gates, mask=low8)
                plsc.store_scatter(idx_v, [r * _TOP_K + lane], vf, mask=low8)
                return c

            lax.fori_loop(0, rows_per_chunk, do_row, 0, unroll=4)
            pltpu.sync_copy(dense_v,
                            dense_hbm.at[pl.ds(base * _E,
                                               rows_per_chunk * _E)])
            pltpu.sync_copy(idx_v,
                            idx_hbm.at[pl.ds(base * _TOP_K,
                                             rows_per_chunk * _TOP_K)])
            return carry

        lax.fori_loop(0, n_chunks, do_chunk, 0)

    return sc_topk


@jax.jit
def _router(x, W_gate):
    T = x.shape[0]
    E = W_gate.shape[1]
    logits = _logits_tc(x, W_gate)
    sc = _make_sc_topk(T, 512)
    dense_flat, idx_flat = sc(logits.reshape(T * E))
    return dense_flat.reshape(T, E), idx_flat.reshape(T, _TOP_K)


def kernel(x, W_gate):
    dense_gates, topk_idx = _router(x, W_gate)
    return dense_gates, topk_idx


# speedup vs baseline: 1.6510x; 1.6510x over previous
import functools

import jax
import jax.numpy as jnp
from jax.experimental import pallas as pl
from jax.experimental.pallas import tpu as pltpu

_TOP_K = 8


def _router_body(x_ref, w_ref, dense_ref, idx_ref, lg_ref):
    logits = lg_ref[...]
    B, E = logits.shape
    rcol = jax.lax.broadcasted_iota(jnp.int32, (B, E), 1).astype(jnp.float32)
    rcol = jnp.float32(E - 1) - rcol
    work = logits
    v0 = None
    idx_cols = []
    for k in range(_TOP_K):
        m = jnp.max(work, axis=1, keepdims=True)
        rsel = jnp.max(jnp.where(work == m, rcol, -1.0), axis=1, keepdims=True)
        if k == 0:
            v0 = m
        idx_cols.append(rsel)
        work = jnp.where(rcol == rsel, -jnp.inf, work)
    expall = jnp.where(work == -jnp.inf, jnp.exp(logits - v0), 0.0)
    ssum = jnp.sum(expall, axis=1, keepdims=True)
    dense_ref[...] = expall / ssum
    idx = jnp.float32(E - 1) - jnp.concatenate(idx_cols, axis=1)
    idx_ref[...] = idx.astype(jnp.int32)
    lg_ref[...] = jnp.dot(x_ref[...], w_ref[...],
                          preferred_element_type=jnp.float32)


@functools.partial(jax.jit, static_argnames=("block_t",))
def _router(x, W_gate, block_t=1024):
    T, D = x.shape
    E = W_gate.shape[1]
    nb = T // block_t
    return pl.pallas_call(
        _router_body,
        grid=(nb + 1,),
        in_specs=[
            pl.BlockSpec((block_t, D), lambda i: (jnp.minimum(i, nb - 1), 0)),
            pl.BlockSpec((D, E), lambda i: (0, 0)),
        ],
        out_specs=[
            pl.BlockSpec((block_t, E), lambda i: (jnp.maximum(i - 1, 0), 0)),
            pl.BlockSpec((block_t, _TOP_K), lambda i: (jnp.maximum(i - 1, 0), 0)),
        ],
        out_shape=[
            jax.ShapeDtypeStruct((T, E), jnp.float32),
            jax.ShapeDtypeStruct((T, _TOP_K), jnp.int32),
        ],
        scratch_shapes=[pltpu.VMEM((block_t, E), jnp.float32)],
        compiler_params=pltpu.CompilerParams(
            dimension_semantics=("arbitrary",),
        ),
    )(x, W_gate)


def kernel(x, W_gate):
    dense_gates, topk_idx = _router(x, W_gate)
    return dense_gates, topk_idx
